# baseline (speedup 1.0000x reference)
import functools

import jax
import jax.numpy as jnp
from jax import lax
from jax.experimental import pallas as pl
from jax.experimental.pallas import tpu as pltpu
from jax.experimental.pallas import tpu_sc as plsc

_N = 10000
_E = 320000
_D = 128
_H = 256
_NPAD = 10240
_NC = 2
_NS = 16
_RPT = _NPAD // _NS
_C = 80
_BN = 1000
_GRID = _N // _BN


def _sc_mesh():
    return plsc.VectorSubcoreMesh(core_axis_name="c", subcore_axis_name="s")


_NB = 4
_G = _NB * _C
_EPAD = 327680
_TRASH = _NPAD - 1


def _agg_pipeline(table_hbm, sidx_hbm, didx_hbm, w_sidx, s_didx, ng,
                  rows, gsems, sidxg, ssems, didxg, dsems, scatter):
    def idxload(g, S):
        return (pltpu.make_async_copy(sidx_hbm.at[w_sidx, g], sidxg[S],
                                      ssems[S]),
                pltpu.make_async_copy(didx_hbm.at[s_didx, g], didxg[S],
                                      dsems[S]))

    def gather(g, b, S):
        return pltpu.make_async_copy(table_hbm.at[sidxg[S].at[b]], rows[b],
                                     gsems[b])

    for d in idxload(0, 0):
        d.start()
    for d in idxload(1, 1):
        d.start()
    for d in idxload(0, 0):
        d.wait()
    for b in range(_NB):
        gather(0, b, 0).start()

    def group_step(g, S, fire_next_idx):
        Sn = 1 - S
        for d in idxload(g + 1, Sn):
            d.wait()
        for b in range(_NB):
            gather(g, b, S).wait()
            scatter(didxg[S].at[b], b)
            gather(g + 1, b, Sn).start()
        if fire_next_idx:
            for d in idxload(g + 2, S):
                d.start()

    def body(t, carry):
        group_step(2 * t, 0, True)
        group_step(2 * t + 1, 1, True)
        return carry

    lax.fori_loop(0, ng // 2 - 1, body, 0)
    group_step(ng - 2, 0, False)
    for b in range(_NB):
        gather(ng - 1, b, 1).wait()
        scatter(didxg[1].at[b], b)


def _agg_l0_body(x_hbm, src_hbm, dst_hbm, zrows_hbm, zdeg_hbm,
                 agg_hbm, deg_hbm,
                 sidxg, didxg, ones, rows, gsems, ssems, dsems, acc, dacc):
    c = lax.axis_index("c")
    s = lax.axis_index("s")
    w = c * _NS + s
    r0 = s * _RPT
    ng = _EPAD // (_NC * _NS * _G)

    pltpu.sync_copy(zrows_hbm, acc.at[pl.ds(r0, _RPT)])
    pltpu.sync_copy(zdeg_hbm, dacc.at[pl.ds(r0, _RPT)])
    for j in range(_C // 16):
        ones[pl.ds(j * 16, 16)] = jnp.ones((16,), jnp.float32)
    plsc.subcore_barrier()

    def scatter(idx_row, b):
        pltpu.sync_copy(rows[b], acc.at[idx_row], add=True)
        pltpu.sync_copy(ones, dacc.at[idx_row], add=True)

    _agg_pipeline(x_hbm, src_hbm, dst_hbm, w, w, ng,
                  rows, gsems, sidxg, ssems, didxg, dsems, scatter)

    plsc.subcore_barrier()
    o0 = c * _NPAD + r0
    pltpu.sync_copy(acc.at[pl.ds(r0, _RPT)], agg_hbm.at[pl.ds(o0, _RPT)])
    pltpu.sync_copy(dacc.at[pl.ds(r0, _RPT)], deg_hbm.at[pl.ds(o0, _RPT)])


def _agg_l0_call(x, src_r, dst_r, zrows, zdeg):
    return pl.kernel(
        _agg_l0_body,
        out_type=[
            jax.ShapeDtypeStruct((_NC * _NPAD, _D), jnp.float32),
            jax.ShapeDtypeStruct((_NC * _NPAD,), jnp.float32),
        ],
        mesh=_sc_mesh(),
        scratch_types=[
            [pltpu.VMEM((_NB, _C), jnp.int32) for _ in range(2)],
            [pltpu.VMEM((_NB, _C), jnp.int32) for _ in range(2)],
            pltpu.VMEM((_C,), jnp.float32),
            [pltpu.VMEM((_C, _D), jnp.float32) for _ in range(_NB)],
            [pltpu.SemaphoreType.DMA for _ in range(_NB)],
            [pltpu.SemaphoreType.DMA for _ in range(2)],
            [pltpu.SemaphoreType.DMA for _ in range(2)],
            pltpu.VMEM_SHARED((_NPAD, _D), jnp.float32),
            pltpu.VMEM_SHARED((_NPAD,), jnp.float32),
        ],
    )(x, src_r, dst_r, zrows, zdeg)


def _agg_h_body(h_hbm, src2_hbm, dst_hbm, zrows_hbm,
                agg_hbm,
                sidxg, didxg, rows, gsems, ssems, dsems, acc):
    c = lax.axis_index("c")
    s = lax.axis_index("s")
    w = c * _NS + s
    r0 = s * _RPT
    ng = _EPAD // (_NS * _G)

    pltpu.sync_copy(zrows_hbm, acc.at[pl.ds(r0, _RPT)])
    plsc.subcore_barrier()

    def scatter(idx_row, b):
        pltpu.sync_copy(rows[b], acc.at[idx_row], add=True)

    _agg_pipeline(h_hbm, src2_hbm, dst_hbm, w, s, ng,
                  rows, gsems, sidxg, ssems, didxg, dsems, scatter)

    plsc.subcore_barrier()
    pltpu.sync_copy(acc.at[pl.ds(r0, _RPT)],
                    agg_hbm.at[pl.ds(c * _NPAD + r0, _RPT)])


def _agg_h_call(h_stacked, src2_r, dst_r, zrows):
    return pl.kernel(
        _agg_h_body,
        out_type=jax.ShapeDtypeStruct((_NC * _NPAD, _D), jnp.float32),
        mesh=_sc_mesh(),
        scratch_types=[
            [pltpu.VMEM((_NB, _C), jnp.int32) for _ in range(2)],
            [pltpu.VMEM((_NB, _C), jnp.int32) for _ in range(2)],
            [pltpu.VMEM((_C, _D), jnp.float32) for _ in range(_NB)],
            [pltpu.SemaphoreType.DMA for _ in range(_NB)],
            [pltpu.SemaphoreType.DMA for _ in range(2)],
            [pltpu.SemaphoreType.DMA for _ in range(2)],
            pltpu.VMEM_SHARED((_NPAD, _D), jnp.float32),
        ],
    )(h_stacked, src2_r, dst_r, zrows)


def _tc_l0_body(x_ref, agg_ref, deg_ref, ws_ref, wn_ref, b_ref,
                hs_ref, invd_ref):
    a = agg_ref[0] + agg_ref[1]
    d = jnp.maximum(deg_ref[0] + deg_ref[1], 1.0)
    invd = 1.0 / d
    z = jnp.dot(x_ref[...], ws_ref[...], preferred_element_type=jnp.float32)
    z += invd * jnp.dot(a, wn_ref[...], preferred_element_type=jnp.float32)
    h = jnp.maximum(z + b_ref[...], 0.0)
    hs_ref[0] = h[:, :_D]
    hs_ref[1] = h[:, _D:]
    invd_ref[...] = invd


def _tc_l0_call(x, aggp, degp, ws, wn, b):
    return pl.pallas_call(
        _tc_l0_body,
        grid=(_GRID,),
        in_specs=[
            pl.BlockSpec((_BN, _D), lambda i: (i, 0)),
            pl.BlockSpec((2, _BN, _D), lambda i: (0, i, 0)),
            pl.BlockSpec((2, _BN, 1), lambda i: (0, i, 0)),
            pl.BlockSpec((_D, _H), lambda i: (0, 0)),
            pl.BlockSpec((_D, _H), lambda i: (0, 0)),
            pl.BlockSpec((1, _H), lambda i: (0, 0)),
        ],
        out_specs=[
            pl.BlockSpec((2, _BN, _D), lambda i: (0, i, 0)),
            pl.BlockSpec((_BN, 1), lambda i: (i, 0)),
        ],
        out_shape=[
            jax.ShapeDtypeStruct((2, _NPAD, _D), jnp.float32),
            jax.ShapeDtypeStruct((_NPAD, 1), jnp.float32),
        ],
    )(x, aggp, degp, ws, wn, b)


def _tc_l_body(hs_ref, agg_ref, invd_ref, ws_ref, wn_ref, b_ref, out_ref):
    ws = ws_ref[...]
    wn = wn_ref[...]
    z = jnp.dot(hs_ref[0], ws[:_D], preferred_element_type=jnp.float32)
    z += jnp.dot(hs_ref[1], ws[_D:], preferred_element_type=jnp.float32)
    za = jnp.dot(agg_ref[0], wn[:_D], preferred_element_type=jnp.float32)
    za += jnp.dot(agg_ref[1], wn[_D:], preferred_element_type=jnp.float32)
    h = jnp.maximum(z + invd_ref[...] * za + b_ref[...], 0.0)
    out_ref[0] = h[:, :_D]
    out_ref[1] = h[:, _D:]


def _tc_l_call(hs, agg, invd, ws, wn, b):
    return pl.pallas_call(
        _tc_l_body,
        grid=(_GRID,),
        in_specs=[
            pl.BlockSpec((2, _BN, _D), lambda i: (0, i, 0)),
            pl.BlockSpec((2, _BN, _D), lambda i: (0, i, 0)),
            pl.BlockSpec((_BN, 1), lambda i: (i, 0)),
            pl.BlockSpec((_H, _H), lambda i: (0, 0)),
            pl.BlockSpec((_H, _H), lambda i: (0, 0)),
            pl.BlockSpec((1, _H), lambda i: (0, 0)),
        ],
        out_specs=pl.BlockSpec((2, _BN, _D), lambda i: (0, i, 0)),
        out_shape=jax.ShapeDtypeStruct((2, _NPAD, _D), jnp.float32),
    )(hs, agg, invd, ws, wn, b)


def _tc_l2_clf_body(hs_ref, agg_ref, invd_ref, ws_ref, wn_ref, b_ref,
                    wc1_ref, bc1_ref, wc2_ref, bc2_ref, out_ref):
    ws = ws_ref[...]
    wn = wn_ref[...]
    z = jnp.dot(hs_ref[0], ws[:_D], preferred_element_type=jnp.float32)
    z += jnp.dot(hs_ref[1], ws[_D:], preferred_element_type=jnp.float32)
    za = jnp.dot(agg_ref[0], wn[:_D], preferred_element_type=jnp.float32)
    za += jnp.dot(agg_ref[1], wn[_D:], preferred_element_type=jnp.float32)
    h = jnp.maximum(z + invd_ref[...] * za + b_ref[...], 0.0)
    hc = jnp.maximum(
        jnp.dot(h, wc1_ref[...], preferred_element_type=jnp.float32)
        + bc1_ref[...], 0.0)
    out_ref[...] = (jnp.dot(hc, wc2_ref[...],
                            preferred_element_type=jnp.float32)
                    + bc2_ref[...])


def _tc_l2_clf_call(hs, agg, invd, ws, wn, b, wc1, bc1, wc2, bc2):
    return pl.pallas_call(
        _tc_l2_clf_body,
        grid=(_GRID,),
        in_specs=[
            pl.BlockSpec((2, _BN, _D), lambda i: (0, i, 0)),
            pl.BlockSpec((2, _BN, _D), lambda i: (0, i, 0)),
            pl.BlockSpec((_BN, 1), lambda i: (i, 0)),
            pl.BlockSpec((_H, _H), lambda i: (0, 0)),
            pl.BlockSpec((_H, _H), lambda i: (0, 0)),
            pl.BlockSpec((1, _H), lambda i: (0, 0)),
            pl.BlockSpec((_H, _H // 2), lambda i: (0, 0)),
            pl.BlockSpec((1, _H // 2), lambda i: (0, 0)),
            pl.BlockSpec((_H // 2, 1), lambda i: (0, 0)),
            pl.BlockSpec((1, 1), lambda i: (0, 0)),
        ],
        out_specs=pl.BlockSpec((_BN, 1), lambda i: (i, 0)),
        out_shape=jax.ShapeDtypeStruct((_N, 1), jnp.float32),
    )(hs, agg, invd, ws, wn, b, wc1, bc1, wc2, bc2)


def kernel(x, edge_index, Wself0, Wneigh0, b0, Wself1, Wneigh1, b1,
           Wself2, Wneigh2, b2, Wc1, bc1, Wc2, bc2):
    src = edge_index[0]
    dst = edge_index[1]
    npad_e = _EPAD - _E
    src_p = jnp.concatenate([src, jnp.zeros((npad_e,), jnp.int32)])
    dst_p = jnp.concatenate([dst, jnp.full((npad_e,), _TRASH, jnp.int32)])
    src2_p = jnp.concatenate([src_p, src_p + _NPAD])
    zrows = jnp.zeros((_RPT, _D), jnp.float32)
    zdeg = jnp.zeros((_RPT,), jnp.float32)

    ng0 = _EPAD // (_NC * _NS * _G)
    ngh = _EPAD // (_NS * _G)
    src_r = src_p.reshape(_NC * _NS, ng0, _NB, _C)
    dst_r0 = dst_p.reshape(_NC * _NS, ng0, _NB, _C)
    src2_r = src2_p.reshape(_NC * _NS, ngh, _NB, _C)
    dst_rh = dst_p.reshape(_NS, ngh, _NB, _C)

    aggp, degp = _agg_l0_call(x, src_r, dst_r0, zrows, zdeg)
    hs1, invd = _tc_l0_call(
        x,
        aggp.reshape(_NC, _NPAD, _D),
        degp.reshape(_NC, _NPAD, 1),
        Wself0, Wneigh0, b0.reshape(1, _H))

    agg1 = _agg_h_call(hs1.reshape(_NC * _NPAD, _D), src2_r, dst_rh, zrows)
    hs2 = _tc_l_call(hs1, agg1.reshape(_NC, _NPAD, _D), invd,
                     Wself1, Wneigh1, b1.reshape(1, _H))

    agg2 = _agg_h_call(hs2.reshape(_NC * _NPAD, _D), src2_r, dst_rh, zrows)
    logits = _tc_l2_clf_call(hs2, agg2.reshape(_NC, _NPAD, _D), invd,
                             Wself2, Wneigh2, b2.reshape(1, _H),
                             Wc1, bc1.reshape(1, _H // 2),
                             Wc2, bc2.reshape(1, 1))
    return logits


# speedup vs baseline: 2.7836x; 2.7836x over previous
import functools

import jax
import jax.numpy as jnp
from jax import lax
from jax.experimental import pallas as pl
from jax.experimental.pallas import tpu as pltpu
from jax.experimental.pallas import tpu_sc as plsc

_N = 10000
_E = 320000
_D = 128
_H = 256
_NPAD = 10240
_NC = 2
_NS = 16
_RPT = _NPAD // _NS
_C = 80
_BN = 1000
_GRID = _N // _BN


def _sc_mesh():
    return plsc.VectorSubcoreMesh(core_axis_name="c", subcore_axis_name="s")


_NB = 4
_G = _NB * _C
_EPAD = 327680


def _agg_pipeline(table_hbm, sidx_hbm, didx_hbm, w_sidx, s_didx, ng,
                  rows, gsems, sidxg, ssems, didxg, dsems, scatter):
    def idxload(g, S):
        return (pltpu.make_async_copy(sidx_hbm.at[w_sidx, g], sidxg[S],
                                      ssems[S]),
                pltpu.make_async_copy(didx_hbm.at[s_didx, g], didxg[S],
                                      dsems[S]))

    def gather(g, b, S):
        return pltpu.make_async_copy(table_hbm.at[sidxg[S].at[b]], rows[b],
                                     gsems[b])

    for d in idxload(0, 0):
        d.start()
    for d in idxload(1, 1):
        d.start()
    for d in idxload(0, 0):
        d.wait()
    for b in range(_NB):
        gather(0, b, 0).start()

    def group_step(g, S, fire_next_idx):
        Sn = 1 - S
        for d in idxload(g + 1, Sn):
            d.wait()
        for b in range(_NB):
            gather(g, b, S).wait()
            scatter(didxg[S].at[b], b)
            gather(g + 1, b, Sn).start()
        if fire_next_idx:
            for d in idxload(g + 2, S):
                d.start()

    def body(t, carry):
        group_step(2 * t, 0, True)
        group_step(2 * t + 1, 1, True)
        return carry

    lax.fori_loop(0, ng // 2 - 1, body, 0)
    group_step(ng - 2, 0, False)
    for b in range(_NB):
        gather(ng - 1, b, 1).wait()
        scatter(didxg[1].at[b], b)


def _agg_l0_body(x_hbm, src_hbm, dst_hbm, zrows_hbm, zdeg_hbm,
                 agg_hbm, deg_hbm,
                 sidxg, didxg, ones, rows, gsems, ssems, dsems, acc, dacc):
    c = lax.axis_index("c")
    s = lax.axis_index("s")
    w = c * _NS + s
    r0 = s * _RPT
    ng = _EPAD // (_NC * _NS * _G)

    pltpu.sync_copy(zrows_hbm, acc.at[pl.ds(r0, _RPT)])
    pltpu.sync_copy(zdeg_hbm, dacc.at[pl.ds(r0, _RPT)])
    for j in range(_C // 16):
        ones[pl.ds(j * 16, 16)] = jnp.ones((16,), jnp.float32)
    plsc.subcore_barrier()

    def scatter(idx_row, b):
        pltpu.sync_copy(rows[b], acc.at[idx_row], add=True)
        pltpu.sync_copy(ones, dacc.at[idx_row], add=True)

    _agg_pipeline(x_hbm, src_hbm, dst_hbm, w, w, ng,
                  rows, gsems, sidxg, ssems, didxg, dsems, scatter)

    plsc.subcore_barrier()
    o0 = c * _NPAD + r0
    pltpu.sync_copy(acc.at[pl.ds(r0, _RPT)], agg_hbm.at[pl.ds(o0, _RPT)])
    pltpu.sync_copy(dacc.at[pl.ds(r0, _RPT)], deg_hbm.at[pl.ds(o0, _RPT)])


def _agg_l0_call(x, src_r, dst_r, zrows, zdeg):
    return pl.kernel(
        _agg_l0_body,
        out_type=[
            jax.ShapeDtypeStruct((_NC * _NPAD, _D), jnp.float32),
            jax.ShapeDtypeStruct((_NC * _NPAD,), jnp.float32),
        ],
        mesh=_sc_mesh(),
        scratch_types=[
            [pltpu.VMEM((_NB, _C), jnp.int32) for _ in range(2)],
            [pltpu.VMEM((_NB, _C), jnp.int32) for _ in range(2)],
            pltpu.VMEM((_C,), jnp.float32),
            [pltpu.VMEM((_C, _D), jnp.float32) for _ in range(_NB)],
            [pltpu.SemaphoreType.DMA for _ in range(_NB)],
            [pltpu.SemaphoreType.DMA for _ in range(2)],
            [pltpu.SemaphoreType.DMA for _ in range(2)],
            pltpu.VMEM_SHARED((_NPAD, _D), jnp.float32),
            pltpu.VMEM_SHARED((_NPAD,), jnp.float32),
        ],
    )(x, src_r, dst_r, zrows, zdeg)


def _agg_h_body(h_hbm, src2_hbm, dst_hbm, zrows_hbm,
                agg_hbm,
                sidxg, didxg, rows, gsems, ssems, dsems, acc):
    c = lax.axis_index("c")
    s = lax.axis_index("s")
    w = c * _NS + s
    r0 = s * _RPT
    ng = _EPAD // (_NS * _G)

    pltpu.sync_copy(zrows_hbm, acc.at[pl.ds(r0, _RPT)])
    plsc.subcore_barrier()

    def scatter(idx_row, b):
        pltpu.sync_copy(rows[b], acc.at[idx_row], add=True)

    _agg_pipeline(h_hbm, src2_hbm, dst_hbm, w, s, ng,
                  rows, gsems, sidxg, ssems, didxg, dsems, scatter)

    plsc.subcore_barrier()
    pltpu.sync_copy(acc.at[pl.ds(r0, _RPT)],
                    agg_hbm.at[pl.ds(c * _NPAD + r0, _RPT)])


def _agg_h_call(h_stacked, src2_r, dst_r, zrows):
    return pl.kernel(
        _agg_h_body,
        out_type=jax.ShapeDtypeStruct((_NC * _NPAD, _D), jnp.float32),
        mesh=_sc_mesh(),
        scratch_types=[
            [pltpu.VMEM((_NB, _C), jnp.int32) for _ in range(2)],
            [pltpu.VMEM((_NB, _C), jnp.int32) for _ in range(2)],
            [pltpu.VMEM((_C, _D), jnp.float32) for _ in range(_NB)],
            [pltpu.SemaphoreType.DMA for _ in range(_NB)],
            [pltpu.SemaphoreType.DMA for _ in range(2)],
            [pltpu.SemaphoreType.DMA for _ in range(2)],
            pltpu.VMEM_SHARED((_NPAD, _D), jnp.float32),
        ],
    )(h_stacked, src2_r, dst_r, zrows)


def _tc_l0_body(x_ref, agg_ref, deg_ref, ws_ref, wn_ref, b_ref,
                hs_ref, invd_ref):
    a = agg_ref[0] + agg_ref[1]
    d = jnp.maximum(deg_ref[0] + deg_ref[1], 1.0)
    invd = 1.0 / d
    z = jnp.dot(x_ref[...], ws_ref[...], preferred_element_type=jnp.float32)
    z += invd * jnp.dot(a, wn_ref[...], preferred_element_type=jnp.float32)
    h = jnp.maximum(z + b_ref[...], 0.0)
    hs_ref[0] = h[:, :_D]
    hs_ref[1] = h[:, _D:]
    invd_ref[...] = invd


def _tc_l0_call(x, aggp, degp, ws, wn, b):
    return pl.pallas_call(
        _tc_l0_body,
        grid=(_GRID,),
        in_specs=[
            pl.BlockSpec((_BN, _D), lambda i: (i, 0)),
            pl.BlockSpec((2, _BN, _D), lambda i: (0, i, 0)),
            pl.BlockSpec((2, _BN, 1), lambda i: (0, i, 0)),
            pl.BlockSpec((_D, _H), lambda i: (0, 0)),
            pl.BlockSpec((_D, _H), lambda i: (0, 0)),
            pl.BlockSpec((1, _H), lambda i: (0, 0)),
        ],
        out_specs=[
            pl.BlockSpec((2, _BN, _D), lambda i: (0, i, 0)),
            pl.BlockSpec((_BN, 1), lambda i: (i, 0)),
        ],
        out_shape=[
            jax.ShapeDtypeStruct((2, _NPAD, _D), jnp.float32),
            jax.ShapeDtypeStruct((_NPAD, 1), jnp.float32),
        ],
    )(x, aggp, degp, ws, wn, b)


def _tc_l_body(hs_ref, agg_ref, invd_ref, ws_ref, wn_ref, b_ref, out_ref):
    ws = ws_ref[...]
    wn = wn_ref[...]
    z = jnp.dot(hs_ref[0], ws[:_D], preferred_element_type=jnp.float32)
    z += jnp.dot(hs_ref[1], ws[_D:], preferred_element_type=jnp.float32)
    za = jnp.dot(agg_ref[0], wn[:_D], preferred_element_type=jnp.float32)
    za += jnp.dot(agg_ref[1], wn[_D:], preferred_element_type=jnp.float32)
    h = jnp.maximum(z + invd_ref[...] * za + b_ref[...], 0.0)
    out_ref[0] = h[:, :_D]
    out_ref[1] = h[:, _D:]


def _tc_l_call(hs, agg, invd, ws, wn, b):
    return pl.pallas_call(
        _tc_l_body,
        grid=(_GRID,),
        in_specs=[
            pl.BlockSpec((2, _BN, _D), lambda i: (0, i, 0)),
            pl.BlockSpec((2, _BN, _D), lambda i: (0, i, 0)),
            pl.BlockSpec((_BN, 1), lambda i: (i, 0)),
            pl.BlockSpec((_H, _H), lambda i: (0, 0)),
            pl.BlockSpec((_H, _H), lambda i: (0, 0)),
            pl.BlockSpec((1, _H), lambda i: (0, 0)),
        ],
        out_specs=pl.BlockSpec((2, _BN, _D), lambda i: (0, i, 0)),
        out_shape=jax.ShapeDtypeStruct((2, _NPAD, _D), jnp.float32),
    )(hs, agg, invd, ws, wn, b)


def _tc_l2_clf_body(hs_ref, agg_ref, invd_ref, ws_ref, wn_ref, b_ref,
                    wc1_ref, bc1_ref, wc2_ref, bc2_ref, out_ref):
    ws = ws_ref[...]
    wn = wn_ref[...]
    z = jnp.dot(hs_ref[0], ws[:_D], preferred_element_type=jnp.float32)
    z += jnp.dot(hs_ref[1], ws[_D:], preferred_element_type=jnp.float32)
    za = jnp.dot(agg_ref[0], wn[:_D], preferred_element_type=jnp.float32)
    za += jnp.dot(agg_ref[1], wn[_D:], preferred_element_type=jnp.float32)
    h = jnp.maximum(z + invd_ref[...] * za + b_ref[...], 0.0)
    hc = jnp.maximum(
        jnp.dot(h, wc1_ref[...], preferred_element_type=jnp.float32)
        + bc1_ref[...], 0.0)
    out_ref[...] = (jnp.dot(hc, wc2_ref[...],
                            preferred_element_type=jnp.float32)
                    + bc2_ref[...])


def _tc_l2_clf_call(hs, agg, invd, ws, wn, b, wc1, bc1, wc2, bc2):
    return pl.pallas_call(
        _tc_l2_clf_body,
        grid=(_GRID,),
        in_specs=[
            pl.BlockSpec((2, _BN, _D), lambda i: (0, i, 0)),
            pl.BlockSpec((2, _BN, _D), lambda i: (0, i, 0)),
            pl.BlockSpec((_BN, 1), lambda i: (i, 0)),
            pl.BlockSpec((_H, _H), lambda i: (0, 0)),
            pl.BlockSpec((_H, _H), lambda i: (0, 0)),
            pl.BlockSpec((1, _H), lambda i: (0, 0)),
            pl.BlockSpec((_H, _H // 2), lambda i: (0, 0)),
            pl.BlockSpec((1, _H // 2), lambda i: (0, 0)),
            pl.BlockSpec((_H // 2, 1), lambda i: (0, 0)),
            pl.BlockSpec((1, 1), lambda i: (0, 0)),
        ],
        out_specs=pl.BlockSpec((_BN, 1), lambda i: (i, 0)),
        out_shape=jax.ShapeDtypeStruct((_N, 1), jnp.float32),
    )(hs, agg, invd, ws, wn, b, wc1, bc1, wc2, bc2)


def kernel(x, edge_index, Wself0, Wneigh0, b0, Wself1, Wneigh1, b1,
           Wself2, Wneigh2, b2, Wc1, bc1, Wc2, bc2):
    src = edge_index[0]
    dst = edge_index[1]
    npad_e = _EPAD - _E
    pad_ids = jax.lax.iota(jnp.int32, npad_e)
    src_p = jnp.concatenate([src, pad_ids % _N])
    dst_p = jnp.concatenate([dst, _N + pad_ids % (_NPAD - _N)])
    src2_p = jnp.concatenate([src_p, src_p + _NPAD])
    zrows = jnp.zeros((_RPT, _D), jnp.float32)
    zdeg = jnp.zeros((_RPT,), jnp.float32)

    ng0 = _EPAD // (_NC * _NS * _G)
    ngh = _EPAD // (_NS * _G)
    src_r = src_p.reshape(_NC * _NS, ng0, _NB, _C)
    dst_r0 = dst_p.reshape(_NC * _NS, ng0, _NB, _C)
    src2_r = src2_p.reshape(_NC * _NS, ngh, _NB, _C)
    dst_rh = dst_p.reshape(_NS, ngh, _NB, _C)

    aggp, degp = _agg_l0_call(x, src_r, dst_r0, zrows, zdeg)
    hs1, invd = _tc_l0_call(
        x,
        aggp.reshape(_NC, _NPAD, _D),
        degp.reshape(_NC, _NPAD, 1),
        Wself0, Wneigh0, b0.reshape(1, _H))

    agg1 = _agg_h_call(hs1.reshape(_NC * _NPAD, _D), src2_r, dst_rh, zrows)
    hs2 = _tc_l_call(hs1, agg1.reshape(_NC, _NPAD, _D), invd,
                     Wself1, Wneigh1, b1.reshape(1, _H))

    agg2 = _agg_h_call(hs2.reshape(_NC * _NPAD, _D), src2_r, dst_rh, zrows)
    logits = _tc_l2_clf_call(hs2, agg2.reshape(_NC, _NPAD, _D), invd,
                             Wself2, Wneigh2, b2.reshape(1, _H),
                             Wc1, bc1.reshape(1, _H // 2),
                             Wc2, bc2.reshape(1, 1))
    return logits
